# baseline (speedup 1.0000x reference)
import jax
import jax.numpy as jnp
from jax import lax
from jax.experimental import pallas as pl
from jax.experimental.pallas import tpu as pltpu
from jax.experimental.pallas import tpu_sc as plsc

_H_TC = 48
_BM1 = 8
_BM2 = 4
_SC_WC = 4


def _tc_sumsq_body(x_ref, o_ref):
    @pl.when(pl.program_id(1) == 0)
    def _init():
        o_ref[...] = jnp.zeros_like(o_ref)

    xb = x_ref[...].reshape(-1, x_ref.shape[-1])
    o_ref[0] += jnp.sum(xb * xb, axis=0, keepdims=True)


def _sc_sumsq_body(x_hbm, out_hbm, buf0, buf1, acc_v, sem0, sem1):
    c = lax.axis_index("c")
    s = lax.axis_index("s")
    wid = s * 2 + c
    b = wid // 16
    h = _H_TC + lax.rem(wid, 16)

    n_chunks = 64 // _SC_WC
    bufs = (buf0, buf1)
    sems = (sem0, sem1)

    def start(k, slot):
        return pltpu.async_copy(
            x_hbm.at[b, h, pl.ds(k * _SC_WC, _SC_WC)], bufs[slot], sems[slot])

    start(0, 0)
    zero = jnp.zeros((16,), jnp.float32)
    acc = (zero, zero, zero, zero, zero, zero)

    def chunk_rows(buf, acc):
        def row_body(r, carry):
            i = r // 64
            rr = lax.rem(r, 64)
            out = []
            for j in range(6):
                v = buf[i, rr, pl.ds(16 * j, 16)]
                out.append(carry[j] + v * v)
            return tuple(out)
        return lax.fori_loop(0, _SC_WC * 64, row_body, acc)

    for k in range(n_chunks):
        slot = k % 2
        pltpu.make_async_copy(
            x_hbm.at[b, h, pl.ds(k * _SC_WC, _SC_WC)], bufs[slot],
            sems[slot]).wait()
        if k + 1 < n_chunks:
            start(k + 1, 1 - slot)
        acc = chunk_rows(bufs[slot], acc)

    for j in range(6):
        acc_v[pl.ds(16 * j, 16)] = acc[j]
    pltpu.sync_copy(acc_v, out_hbm.at[wid])


def _norm_body(ptc_ref, psc_ref, gamma_ref, scale_ref):
    gsq = ptc_ref[0] + jnp.sum(psc_ref[...], axis=0, keepdims=True)
    gx = jnp.sqrt(gsq)
    mean = jnp.mean(gx)
    scale_ref[0] = gamma_ref[...] * (gx / (mean + 1e-6)) + 1.0


def _apply_body(scale_ref, beta_ref, x_ref, o_ref):
    scale = scale_ref[0].reshape(1, 1, 1, 1, -1)
    o_ref[...] = scale * x_ref[...] + beta_ref[...].reshape(1, 1, 1, 1, -1)


def kernel(x, gamma, beta):
    B, H, W, D, C = x.shape

    sc_partial = pl.kernel(
        _sc_sumsq_body,
        out_type=jax.ShapeDtypeStruct((32, C), jnp.float32),
        mesh=plsc.VectorSubcoreMesh(core_axis_name="c", subcore_axis_name="s"),
        scratch_types=[
            pltpu.VMEM((_SC_WC, D, C), jnp.float32),
            pltpu.VMEM((_SC_WC, D, C), jnp.float32),
            pltpu.VMEM((C,), jnp.float32),
            pltpu.SemaphoreType.DMA,
            pltpu.SemaphoreType.DMA,
        ],
        compiler_params=pltpu.CompilerParams(use_tc_tiling_on_sc=True),
    )(x)

    tc_partial = pl.pallas_call(
        _tc_sumsq_body,
        grid=(B, _H_TC // _BM1),
        in_specs=[
            pl.BlockSpec((1, _BM1, W, D, C), lambda b, i: (b, i, 0, 0, 0))],
        out_specs=pl.BlockSpec((1, 1, C), lambda b, i: (b, 0, 0)),
        out_shape=jax.ShapeDtypeStruct((B, 1, C), jnp.float32),
        compiler_params=pltpu.CompilerParams(
            dimension_semantics=("parallel", "arbitrary")),
    )(x)

    scale = pl.pallas_call(
        _norm_body,
        grid=(B,),
        in_specs=[
            pl.BlockSpec((1, 1, C), lambda b: (b, 0, 0)),
            pl.BlockSpec((16, C), lambda b: (b, 0)),
            pl.BlockSpec((1, C), lambda b: (0, 0)),
        ],
        out_specs=pl.BlockSpec((1, 1, C), lambda b: (b, 0, 0)),
        out_shape=jax.ShapeDtypeStruct((B, 1, C), jnp.float32),
    )(tc_partial, sc_partial, gamma)

    out = pl.pallas_call(
        _apply_body,
        grid=(B, H // _BM2),
        in_specs=[
            pl.BlockSpec((1, 1, C), lambda b, i: (b, 0, 0)),
            pl.BlockSpec((1, C), lambda b, i: (0, 0)),
            pl.BlockSpec((1, _BM2, W, D, C), lambda b, i: (b, i, 0, 0, 0)),
        ],
        out_specs=pl.BlockSpec((1, _BM2, W, D, C), lambda b, i: (b, i, 0, 0, 0)),
        out_shape=jax.ShapeDtypeStruct((B, H, W, D, C), jnp.float32),
        compiler_params=pltpu.CompilerParams(
            dimension_semantics=("parallel", "parallel")),
    )(scale, beta, x)

    return out
